# Initial kernel scaffold; baseline (speedup 1.0000x reference)
import jax
import jax.numpy as jnp
from jax.experimental import pallas as pl


def kernel(input, pre_edge_feat, adj, degree, W_fc, W0, b0, W1, b1, W2, b2):
    raise NotImplementedError("write your pallas kernel here")


# speedup vs baseline: 4.3459x; 4.3459x over previous
import functools

import jax
import jax.numpy as jnp
from jax import lax
from jax.experimental import pallas as pl
from jax.experimental.pallas import tpu as pltpu
from jax.experimental.pallas import tpu_sc as plsc

N = 10000
E = 320000
D = 128
HID = 32
NC = 2
NS = 16
NW = NC * NS
NPAD = 10240
SL = NPAD // NS
EPAD = 327680
EW = EPAD // NW
CHUNK = 128
NCHUNK = EW // CHUNK

_HIGH = lax.Precision.HIGHEST


def _dense_pre_body(inp, wfc, w0, b0, deg, x_o, af_o, newx_o, rdh_o, rdeg_o):
    a = inp[...]
    x = lax.dot_general(a, wfc[...], (((1,), (1,)), ((), ())), precision=_HIGH)
    af = lax.dot_general(a, w0[...], (((1,), (1,)), ((), ())), precision=_HIGH)
    af = af + b0[...]
    d = deg[...]
    dh = jnp.sqrt(d)
    x_o[...] = x
    af_o[...] = jnp.concatenate(
        [af, jnp.zeros((af.shape[0], D - HID), jnp.float32)], axis=1)
    newx_o[...] = x / dh
    rdh_o[...] = 1.0 / dh
    rdeg_o[...] = 1.0 / d


def _dense_pre(inp, wfc, w0, b0, deg):
    blk = 1000
    grid = N // blk
    return pl.pallas_call(
        _dense_pre_body,
        grid=(grid,),
        in_specs=[
            pl.BlockSpec((blk, D), lambda i: (i, 0)),
            pl.BlockSpec((D, D), lambda i: (0, 0)),
            pl.BlockSpec((HID, D), lambda i: (0, 0)),
            pl.BlockSpec((1, HID), lambda i: (0, 0)),
            pl.BlockSpec((blk, 1), lambda i: (i, 0)),
        ],
        out_specs=[
            pl.BlockSpec((blk, D), lambda i: (i, 0)),
            pl.BlockSpec((blk, D), lambda i: (i, 0)),
            pl.BlockSpec((blk, D), lambda i: (i, 0)),
            pl.BlockSpec((blk, 1), lambda i: (i, 0)),
            pl.BlockSpec((blk, 1), lambda i: (i, 0)),
        ],
        out_shape=[
            jax.ShapeDtypeStruct((N, D), jnp.float32),
            jax.ShapeDtypeStruct((N, D), jnp.float32),
            jax.ShapeDtypeStruct((N, D), jnp.float32),
            jax.ShapeDtypeStruct((N, 1), jnp.float32),
            jax.ShapeDtypeStruct((N, 1), jnp.float32),
        ],
    )(inp, wfc, w0, b0, deg)


def _edge_mlp_body(l_r, r_r, w1, b1, w2, b2, s_o):
    l = l_r[...][:, :HID]
    r = r_r[...][:, :HID]
    feat = jnp.concatenate([l, r, jnp.abs(l - r)], axis=1)
    h = lax.dot_general(feat, w1[...], (((1,), (1,)), ((), ())), precision=_HIGH)
    h = jnp.maximum(h + b1[...], 0.0)
    z = jnp.sum(h * w2[...], axis=1, keepdims=True) + b2[0, 0]
    s_o[...] = 1.0 / (1.0 + jnp.exp(-z))


def _edge_mlp(l, r, w1, b1, w2, b2):
    blk = 2048
    grid = EPAD // blk
    return pl.pallas_call(
        _edge_mlp_body,
        grid=(grid,),
        in_specs=[
            pl.BlockSpec((blk, D), lambda i: (i, 0)),
            pl.BlockSpec((blk, D), lambda i: (i, 0)),
            pl.BlockSpec((HID, 3 * HID), lambda i: (0, 0)),
            pl.BlockSpec((1, HID), lambda i: (0, 0)),
            pl.BlockSpec((1, HID), lambda i: (0, 0)),
            pl.BlockSpec((1, 1), lambda i: (0, 0)),
        ],
        out_specs=pl.BlockSpec((blk, 1), lambda i: (i, 0)),
        out_shape=jax.ShapeDtypeStruct((EPAD, 1), jnp.float32),
    )(l, r, w1, b1, w2, b2)


def _final_aggr_body(p0, p1, x_r, pef, rdh, out):
    a = (p0[...] + p1[...]) * rdh[...]
    p = pef[...]
    out[...] = a * p + x_r[...] * (1.0 - p)


def _final_aggr(aggrp, x, pef, rdh):
    blk = 1024
    grid = NPAD // blk
    return pl.pallas_call(
        _final_aggr_body,
        grid=(grid,),
        in_specs=[
            pl.BlockSpec((blk, D), lambda i: (i, 0)),
            pl.BlockSpec((blk, D), lambda i: (i + NPAD // 1024, 0)),
            pl.BlockSpec((blk, D), lambda i: (i, 0)),
            pl.BlockSpec((blk, 1), lambda i: (i, 0)),
            pl.BlockSpec((blk, 1), lambda i: (i, 0)),
        ],
        out_specs=pl.BlockSpec((blk, D), lambda i: (i, 0)),
        out_shape=jax.ShapeDtypeStruct((N, D), jnp.float32),
    )(aggrp, aggrp, x, pef, rdh)


def _edge_out_body(e2p, rdeg, out):
    v = e2p[...]
    e2 = (v[0] + v[1])[None, :N]
    out[...] = rdeg[...] * e2


def _edge_out(e2p, rdeg):
    blk = 200
    grid = N // blk
    return pl.pallas_call(
        _edge_out_body,
        grid=(grid,),
        in_specs=[
            pl.BlockSpec((2, NPAD), lambda i: (0, 0)),
            pl.BlockSpec((blk, 1), lambda i: (i, 0)),
        ],
        out_specs=pl.BlockSpec((blk, N), lambda i: (i, 0)),
        out_shape=jax.ShapeDtypeStruct((N, N), jnp.float32),
    )(e2p, rdeg)


_Z16 = functools.partial(jnp.zeros, (16,), jnp.float32)


def _worker_base():
    c = lax.axis_index("c")
    s = lax.axis_index("s")
    wid = c * NS + s
    return c, s, wid * EW


def _zero_slice(zb, acc, s, width):
    n = zb.shape[0]
    if len(zb.shape) == 1:
        for j in range(n // 16):
            zb[pl.ds(j * 16, 16)] = _Z16()
        @pl.loop(0, width // n)
        def _(r):
            pltpu.sync_copy(zb, acc.at[pl.ds(s * width + r * n, n)])
    else:
        for i in range(n):
            for j in range(zb.shape[1] // 16):
                zb[i, pl.ds(j * 16, 16)] = _Z16()
        @pl.loop(0, width // n)
        def _(r):
            pltpu.sync_copy(zb, acc.at[pl.ds(s * width + r * n, n)])


def _edge_gather_body(rowp, colp, af, l_o, r_o, idxa, idxb, lv, rv, sem):
    _, _, base = _worker_base()

    @pl.loop(0, NCHUNK)
    def _(k):
        off = pl.multiple_of(base + k * CHUNK, CHUNK)
        pltpu.sync_copy(rowp.at[pl.ds(off, CHUNK)], idxa)
        pltpu.sync_copy(colp.at[pl.ds(off, CHUNK)], idxb)
        pltpu.async_copy(af.at[idxa], lv, sem).wait()
        pltpu.async_copy(af.at[idxb], rv, sem).wait()
        pltpu.sync_copy(lv, l_o.at[pl.ds(off, CHUNK)])
        pltpu.sync_copy(rv, r_o.at[pl.ds(off, CHUNK)])


def _edge_gather(rowp, colp, af_pad):
    mesh = plsc.VectorSubcoreMesh(core_axis_name="c", subcore_axis_name="s")
    f = pl.kernel(
        _edge_gather_body,
        out_type=[
            jax.ShapeDtypeStruct((EPAD, D), jnp.float32),
            jax.ShapeDtypeStruct((EPAD, D), jnp.float32),
        ],
        mesh=mesh,
        scratch_types=[
            pltpu.VMEM((CHUNK,), jnp.int32),
            pltpu.VMEM((CHUNK,), jnp.int32),
            pltpu.VMEM((CHUNK, D), jnp.float32),
            pltpu.VMEM((CHUNK, D), jnp.float32),
            pltpu.SemaphoreType.DMA,
        ],
    )
    return f(rowp, colp, af_pad)


def _seg_e1_body(rowp, sflat, e1p, acc, idxv, valv, zb):
    c, s, base = _worker_base()
    _zero_slice(zb, acc, s, SL)
    plsc.subcore_barrier()

    @pl.loop(0, NCHUNK)
    def _(k):
        off = pl.multiple_of(base + k * CHUNK, CHUNK)
        pltpu.sync_copy(rowp.at[pl.ds(off, CHUNK)], idxv)
        pltpu.sync_copy(sflat.at[pl.ds(off, CHUNK)], valv)
        pltpu.sync_copy(valv, acc.at[idxv], add=True)

    plsc.subcore_barrier()
    pltpu.sync_copy(acc.at[pl.ds(s * SL, SL)],
                    e1p.at[pl.ds(c * NPAD + s * SL, SL)])


def _seg_e1(rowp, sflat):
    mesh = plsc.VectorSubcoreMesh(core_axis_name="c", subcore_axis_name="s")
    f = pl.kernel(
        _seg_e1_body,
        out_type=jax.ShapeDtypeStruct((NC * NPAD,), jnp.float32),
        mesh=mesh,
        scratch_types=[
            pltpu.VMEM_SHARED((NPAD,), jnp.float32),
            pltpu.VMEM((CHUNK,), jnp.int32),
            pltpu.VMEM((CHUNK,), jnp.float32),
            pltpu.VMEM((SL // 4,), jnp.float32),
        ],
    )
    return f(rowp, sflat)


def _seg_e2_body(rowp, colp, e1p, e2p, acc, e1a, e1b, idxv, rowv, valv, zb):
    c, s, base = _worker_base()
    _zero_slice(zb, acc, s, SL)
    pltpu.sync_copy(e1p.at[pl.ds(0, NPAD)], e1a)
    pltpu.sync_copy(e1p.at[pl.ds(NPAD, NPAD)], e1b)
    plsc.subcore_barrier()

    @pl.loop(0, NCHUNK)
    def _(k):
        off = pl.multiple_of(base + k * CHUNK, CHUNK)
        pltpu.sync_copy(colp.at[pl.ds(off, CHUNK)], idxv)
        pltpu.sync_copy(rowp.at[pl.ds(off, CHUNK)], rowv)
        for i in range(CHUNK // 16):
            cv = idxv[pl.ds(i * 16, 16)]
            v = plsc.load_gather(e1a, [cv]) + plsc.load_gather(e1b, [cv])
            valv[pl.ds(i * 16, 16)] = v
        pltpu.sync_copy(valv, acc.at[rowv], add=True)

    plsc.subcore_barrier()
    pltpu.sync_copy(acc.at[pl.ds(s * SL, SL)],
                    e2p.at[pl.ds(c * NPAD + s * SL, SL)])


def _seg_e2(rowp, colp, e1p):
    mesh = plsc.VectorSubcoreMesh(core_axis_name="c", subcore_axis_name="s")
    f = pl.kernel(
        _seg_e2_body,
        out_type=jax.ShapeDtypeStruct((NC * NPAD,), jnp.float32),
        mesh=mesh,
        compiler_params=pltpu.CompilerParams(needs_layout_passes=False),
        scratch_types=[
            pltpu.VMEM_SHARED((NPAD,), jnp.float32),
            pltpu.VMEM((NPAD,), jnp.float32),
            pltpu.VMEM((NPAD,), jnp.float32),
            pltpu.VMEM((CHUNK,), jnp.int32),
            pltpu.VMEM((CHUNK,), jnp.int32),
            pltpu.VMEM((CHUNK,), jnp.float32),
            pltpu.VMEM((SL // 4,), jnp.float32),
        ],
    )
    return f(rowp, colp, e1p)


def _seg_aggr_body(rowp, colp, newx, aggrp, acc, idxv, rowv, rows_v, zb, sem):
    c, s, base = _worker_base()
    _zero_slice(zb, acc, s, SL)
    plsc.subcore_barrier()

    @pl.loop(0, NCHUNK)
    def _(k):
        off = pl.multiple_of(base + k * CHUNK, CHUNK)
        pltpu.sync_copy(colp.at[pl.ds(off, CHUNK)], idxv)
        pltpu.sync_copy(rowp.at[pl.ds(off, CHUNK)], rowv)
        pltpu.async_copy(newx.at[idxv], rows_v, sem).wait()
        pltpu.sync_copy(rows_v, acc.at[rowv], add=True)

    plsc.subcore_barrier()
    pltpu.sync_copy(acc.at[pl.ds(s * SL, SL)],
                    aggrp.at[pl.ds(c * NPAD + s * SL, SL)])


def _seg_aggr(rowp, colp, newx):
    mesh = plsc.VectorSubcoreMesh(core_axis_name="c", subcore_axis_name="s")
    f = pl.kernel(
        _seg_aggr_body,
        out_type=jax.ShapeDtypeStruct((NC * NPAD, D), jnp.float32),
        mesh=mesh,
        scratch_types=[
            pltpu.VMEM_SHARED((NPAD, D), jnp.float32),
            pltpu.VMEM((CHUNK,), jnp.int32),
            pltpu.VMEM((CHUNK,), jnp.int32),
            pltpu.VMEM((CHUNK, D), jnp.float32),
            pltpu.VMEM((16, D), jnp.float32),
            pltpu.SemaphoreType.DMA,
        ],
    )
    return f(rowp, colp, newx)


def kernel(input, pre_edge_feat, adj, degree, W_fc, W0, b0, W1, b1, W2, b2):
    row = adj[0].astype(jnp.int32)
    col = adj[1].astype(jnp.int32)
    npad_e = EPAD - E
    pad_i = jnp.arange(npad_e, dtype=jnp.int32)
    row_p = jnp.concatenate([row, N + (pad_i % (NPAD - N))])
    col_p = jnp.concatenate([col, pad_i % N])

    deg2 = degree.reshape(N, 1)
    b0r = b0.reshape(1, HID)
    b1r = b1.reshape(1, HID)
    b2r = b2.reshape(1, 1)

    x, af, new_x, rdh, rdeg = _dense_pre(input, W_fc, W0, b0r, deg2)
    af_pad = jnp.pad(af, ((0, NPAD - N), (0, 0)))

    l, r = _edge_gather(row_p, col_p, af_pad)
    s = _edge_mlp(l, r, W1, b1r, W2, b2r)

    e1p = _seg_e1(row_p, s.reshape(EPAD))
    e2p = _seg_e2(row_p, col_p, e1p)
    aggrp = _seg_aggr(row_p, col_p, new_x)

    aggr_x = _final_aggr(aggrp.reshape(NC * NPAD, D), x, pre_edge_feat, rdh)
    edge_out = _edge_out(e2p.reshape(NC, NPAD), rdeg)
    return (aggr_x, edge_out)
